# Initial kernel scaffold; baseline (speedup 1.0000x reference)
import jax
import jax.numpy as jnp
from jax.experimental import pallas as pl


def kernel(x, edge_index, edge_attr, W0, b0, W1, b1, W2, b2, W3, b3, We, be, Wc1, bc1, Wc2, bc2):
    raise NotImplementedError("write your pallas kernel here")


# speedup vs baseline: 13.3599x; 13.3599x over previous
import functools

import jax
import jax.numpy as jnp
from jax import lax
from jax.experimental import pallas as pl
from jax.experimental.pallas import tpu as pltpu
from jax.experimental.pallas import tpu_sc as plsc

NC = 2
NS = 16
NW = NC * NS

CW = 80
RPC = 5
RPT = 640
NP = NS * RPT


def _mesh():
  return plsc.VectorSubcoreMesh(
      core_axis_name="c", subcore_axis_name="s", num_cores=NC, num_subcores=NS)


_SC_PARAMS = pltpu.CompilerParams(use_tc_tiling_on_sc=False)


def _sc_degree(dst3):
  rows_per_w = dst3.shape[1]

  @functools.partial(
      pl.kernel,
      out_type=jax.ShapeDtypeStruct((NC, NP, 16), jnp.float32),
      mesh=_mesh(),
      compiler_params=_SC_PARAMS,
      scratch_types=[
          pltpu.VMEM((rows_per_w, CW), jnp.int32),
          pltpu.VMEM((CW, 16), jnp.float32),
          pltpu.VMEM((RPT, 16), jnp.float32),
          pltpu.VMEM_SHARED((NP, 16), jnp.float32),
          pltpu.SemaphoreType.DMA,
      ],
  )
  def deg_kernel(dst_hbm, out_hbm, idx_v, ones_v, bounce_v, deg_sh, sem):
    c = lax.axis_index("c")
    s = lax.axis_index("s")
    w = s * NC + c

    zero16 = jnp.zeros((16,), jnp.float32)
    one16 = jnp.full((16,), 1.0, jnp.float32)

    def zrow(i, carry):
      bounce_v[i, :] = zero16
      return carry

    lax.fori_loop(0, RPT, zrow, 0)

    def orow(i, carry):
      ones_v[i, :] = one16
      return carry

    lax.fori_loop(0, CW, orow, 0)

    pltpu.sync_copy(bounce_v, deg_sh.at[pl.ds(s * RPT, RPT)])
    plsc.subcore_barrier()

    pltpu.sync_copy(dst_hbm.at[w], idx_v)
    for g in range(0, rows_per_w, 25):
      ds_ = [pltpu.async_copy(ones_v, deg_sh.at[idx_v.at[g + k]], sem,
                              add=True) for k in range(25)]
      for d in ds_:
        d.wait()

    plsc.subcore_barrier()
    pltpu.sync_copy(deg_sh.at[pl.ds(s * RPT, RPT)], bounce_v)
    pltpu.sync_copy(bounce_v, out_hbm.at[c].at[pl.ds(s * RPT, RPT)])

  return deg_kernel(dst3)


def _sc_scatter(src3, dst3, table):
  n, h = table.shape
  rows_per_w = src3.shape[1]
  ngrp = rows_per_w // RPC

  @functools.partial(
      pl.kernel,
      out_type=jax.ShapeDtypeStruct((NC, NP, h), jnp.float32),
      mesh=_mesh(),
      compiler_params=_SC_PARAMS,
      scratch_types=[
          pltpu.VMEM((rows_per_w, CW), jnp.int32),
          pltpu.VMEM((rows_per_w, CW), jnp.int32),
          [pltpu.VMEM((CW, h), jnp.float32) for _ in range(2 * RPC)],
          pltpu.VMEM((CW, h), jnp.float32),
          pltpu.VMEM_SHARED((NP, h), jnp.float32),
          pltpu.SemaphoreType.DMA,
          pltpu.SemaphoreType.DMA,
      ],
  )
  def scat_kernel(src_hbm, dst_hbm, tab_hbm, out_hbm, sidx_v, didx_v, rows_v,
                  bounce_v, acc_sh, gsem, ssem):
    c = lax.axis_index("c")
    s = lax.axis_index("s")
    w = s * NC + c

    zero16 = jnp.zeros((16,), jnp.float32)

    def zrow(i, carry):
      for j in range(h // 16):
        bounce_v[i, pl.ds(16 * j, 16)] = zero16
      return carry

    lax.fori_loop(0, CW, zrow, 0)
    for j in range(RPT // CW):
      pltpu.sync_copy(bounce_v, acc_sh.at[pl.ds(s * RPT + j * CW, CW)])
    plsc.subcore_barrier()

    pltpu.sync_copy(src_hbm.at[w], sidx_v)
    pltpu.sync_copy(dst_hbm.at[w], didx_v)

    pend = [None, None]
    for g in range(ngrp):
      bs = g % 2
      if pend[bs] is not None:
        for d in pend[bs]:
          d.wait()
      gds = []
      for j in range(RPC):
        r = g * RPC + j
        gds.append(pltpu.async_copy(
            tab_hbm.at[sidx_v.at[r]], rows_v[bs * RPC + j], gsem))
      for d in gds:
        d.wait()
      sds = []
      for j in range(RPC):
        r = g * RPC + j
        sds.append(pltpu.async_copy(
            rows_v[bs * RPC + j], acc_sh.at[didx_v.at[r]], ssem, add=True))
      pend[bs] = sds
    for p in pend:
      if p is not None:
        for d in p:
          d.wait()

    plsc.subcore_barrier()
    for j in range(RPT // CW):
      r = s * RPT + j * CW
      pltpu.sync_copy(acc_sh.at[pl.ds(r, CW)], bounce_v)
      pltpu.sync_copy(bounce_v, out_hbm.at[c].at[pl.ds(r, CW)])

  return scat_kernel(src3, dst3, table)


def _sc_gather2(src3, dst3, u, v):
  n, h = u.shape
  rows_per_w = src3.shape[1]
  e = NW * rows_per_w * CW
  ngrp = rows_per_w // RPC

  @functools.partial(
      pl.kernel,
      out_type=(jax.ShapeDtypeStruct((e, h), jnp.float32),
                jax.ShapeDtypeStruct((e, h), jnp.float32)),
      mesh=_mesh(),
      compiler_params=_SC_PARAMS,
      scratch_types=[
          pltpu.VMEM((rows_per_w, CW), jnp.int32),
          pltpu.VMEM((rows_per_w, CW), jnp.int32),
          [pltpu.VMEM((CW, h), jnp.float32) for _ in range(2 * RPC)],
          [pltpu.VMEM((CW, h), jnp.float32) for _ in range(2 * RPC)],
          pltpu.SemaphoreType.DMA,
          pltpu.SemaphoreType.DMA,
      ],
  )
  def gat_kernel(src_hbm, dst_hbm, u_hbm, v_hbm, su_hbm, dv_hbm, sidx_v,
                 didx_v, ubuf_v, vbuf_v, gsem, wsem):
    c = lax.axis_index("c")
    s = lax.axis_index("s")
    w = s * NC + c

    pltpu.sync_copy(src_hbm.at[w], sidx_v)
    pltpu.sync_copy(dst_hbm.at[w], didx_v)

    pend = [None, None]
    for g in range(ngrp):
      bs = g % 2
      if pend[bs] is not None:
        for d in pend[bs]:
          d.wait()
      gds = []
      for j in range(RPC):
        r = g * RPC + j
        gds.append(pltpu.async_copy(
            u_hbm.at[sidx_v.at[r]], ubuf_v[bs * RPC + j], gsem))
        gds.append(pltpu.async_copy(
            v_hbm.at[didx_v.at[r]], vbuf_v[bs * RPC + j], gsem))
      for d in gds:
        d.wait()
      wds = []
      for j in range(RPC):
        o = (w * rows_per_w + g * RPC + j) * CW
        wds.append(pltpu.async_copy(
            ubuf_v[bs * RPC + j], su_hbm.at[pl.ds(o, CW)], wsem))
        wds.append(pltpu.async_copy(
            vbuf_v[bs * RPC + j], dv_hbm.at[pl.ds(o, CW)], wsem))
      pend[bs] = wds
    for p in pend:
      if p is not None:
        for d in p:
          d.wait()

  return gat_kernel(src3, dst3, u, v)


def _dinv_from(degp_ref, n):
  deg = degp_ref[0:n, 0:1] + degp_ref[NP:NP + n, 0:1] + 1.0
  return 1.0 / jnp.sqrt(deg)


def _tc_embed(x, w0, b0, w1, degp):
  n = x.shape[0]
  h = w1.shape[1]

  def body(x_ref, w0_ref, b0_ref, w1_ref, degp_ref, out_ref):
    dinv = _dinv_from(degp_ref, n)
    w01 = jnp.dot(w0_ref[...], w1_ref[...], preferred_element_type=jnp.float32)
    b01 = jnp.dot(b0_ref[...], w1_ref[...], preferred_element_type=jnp.float32)
    t = jnp.dot(x_ref[...], w01, preferred_element_type=jnp.float32) + b01
    out_ref[...] = t * dinv

  return pl.pallas_call(
      body,
      out_shape=jax.ShapeDtypeStruct((n, h), jnp.float32),
  )(x, w0, b0.reshape(1, -1), w1, degp)


def _tc_layer(accp, table, degp, b, w_next):
  n, h = table.shape

  def body(accp_ref, tab_ref, degp_ref, b_ref, w_ref, out_ref):
    dinv = _dinv_from(degp_ref, n)
    acc = accp_ref[0:n, :] + accp_ref[NP:NP + n, :] + tab_ref[...]
    hh = jnp.maximum(acc * dinv + b_ref[...], 0.0)
    out_ref[...] = jnp.dot(hh, w_ref[...],
                           preferred_element_type=jnp.float32) * dinv

  return pl.pallas_call(
      body,
      out_shape=jax.ShapeDtypeStruct((n, h), jnp.float32),
  )(accp, table, degp, b.reshape(1, -1), w_next)


def _tc_head(accp, table, degp, b, wc1):
  n, h = table.shape

  def body(accp_ref, tab_ref, degp_ref, b_ref, wc1_ref, u_ref, v_ref):
    dinv = _dinv_from(degp_ref, n)
    acc = accp_ref[0:n, :] + accp_ref[NP:NP + n, :] + tab_ref[...]
    h3 = acc * dinv + b_ref[...]
    u_ref[...] = jnp.dot(h3, wc1_ref[0:h, :],
                         preferred_element_type=jnp.float32)
    v_ref[...] = jnp.dot(h3, wc1_ref[h:2 * h, :],
                         preferred_element_type=jnp.float32)

  return pl.pallas_call(
      body,
      out_shape=(jax.ShapeDtypeStruct((n, h), jnp.float32),
                 jax.ShapeDtypeStruct((n, h), jnp.float32)),
  )(accp, table, degp, b.reshape(1, -1), wc1)


def _tc_final(su, dv, edge_attr, we, be, wc1, bc1, wc2, bc2):
  e, h = su.shape
  fe = edge_attr.shape[1]
  blk = 8000
  grid = e // blk

  def body(su_ref, dv_ref, ea_ref, we_ref, be_ref, wc1e_ref, bc1_ref, wc2_ref,
           bc2_ref, out_ref):
    ef = jnp.maximum(
        jnp.dot(ea_ref[...], we_ref[...], preferred_element_type=jnp.float32)
        + be_ref[...], 0.0)
    t = jnp.dot(ef, wc1e_ref[...], preferred_element_type=jnp.float32)
    z = jnp.maximum(su_ref[...] + dv_ref[...] + t + bc1_ref[...], 0.0)
    logit = jnp.dot(z, wc2_ref[...], preferred_element_type=jnp.float32) \
        + bc2_ref[...]
    out_ref[...] = 1.0 / (1.0 + jnp.exp(-logit))

  return pl.pallas_call(
      body,
      grid=(grid,),
      in_specs=[
          pl.BlockSpec((blk, h), lambda i: (i, 0)),
          pl.BlockSpec((blk, h), lambda i: (i, 0)),
          pl.BlockSpec((blk, fe), lambda i: (i, 0)),
          pl.BlockSpec((fe, h), lambda i: (0, 0)),
          pl.BlockSpec((1, h), lambda i: (0, 0)),
          pl.BlockSpec((h, h), lambda i: (0, 0)),
          pl.BlockSpec((1, h), lambda i: (0, 0)),
          pl.BlockSpec((h, 1), lambda i: (0, 0)),
          pl.BlockSpec((1, 1), lambda i: (0, 0)),
      ],
      out_specs=pl.BlockSpec((blk, 1), lambda i: (i, 0)),
      out_shape=jax.ShapeDtypeStruct((e, 1), jnp.float32),
  )(su, dv, edge_attr, we, be.reshape(1, -1), wc1[2 * h:, :],
    bc1.reshape(1, -1), wc2, bc2.reshape(1, -1))


def kernel(x, edge_index, edge_attr, W0, b0, W1, b1, W2, b2, W3, b3, We, be,
           Wc1, bc1, Wc2, bc2):
  n = x.shape[0]
  e = edge_index.shape[1]
  rows_per_w = e // (NW * CW)
  ei = edge_index.astype(jnp.int32)
  src3 = ei[0].reshape(NW, rows_per_w, CW)
  dst3 = ei[1].reshape(NW, rows_per_w, CW)

  degp = _sc_degree(dst3).reshape(NC * NP, 16)
  t1 = _tc_embed(x, W0, b0, W1, degp)
  acc1 = _sc_scatter(src3, dst3, t1).reshape(NC * NP, -1)
  t2 = _tc_layer(acc1, t1, degp, b1, W2)
  acc2 = _sc_scatter(src3, dst3, t2).reshape(NC * NP, -1)
  t3 = _tc_layer(acc2, t2, degp, b2, W3)
  acc3 = _sc_scatter(src3, dst3, t3).reshape(NC * NP, -1)
  u, v = _tc_head(acc3, t3, degp, b3, Wc1)
  su, dv = _sc_gather2(src3, dst3, u, v)
  p = _tc_final(su, dv, edge_attr, We, be, Wc1, bc1, Wc2, bc2)
  return p[:, 0]


# speedup vs baseline: 20.7267x; 1.5514x over previous
import functools

import jax
import jax.numpy as jnp
from jax import lax
from jax.experimental import pallas as pl
from jax.experimental.pallas import tpu as pltpu
from jax.experimental.pallas import tpu_sc as plsc

NC = 2
NS = 16
NW = NC * NS

CW = 80
RPC = 5
RPT = 640
NP = NS * RPT


def _mesh():
  return plsc.VectorSubcoreMesh(
      core_axis_name="c", subcore_axis_name="s", num_cores=NC, num_subcores=NS)


_SC_PARAMS = pltpu.CompilerParams(use_tc_tiling_on_sc=False)


def _sc_degree(dst3):
  rows_per_w = dst3.shape[1]

  @functools.partial(
      pl.kernel,
      out_type=jax.ShapeDtypeStruct((NC, NP, 16), jnp.float32),
      mesh=_mesh(),
      compiler_params=_SC_PARAMS,
      scratch_types=[
          pltpu.VMEM((rows_per_w, CW), jnp.int32),
          pltpu.VMEM((CW, 16), jnp.float32),
          pltpu.VMEM((RPT, 16), jnp.float32),
          pltpu.VMEM_SHARED((NP, 16), jnp.float32),
          pltpu.SemaphoreType.DMA,
      ],
  )
  def deg_kernel(dst_hbm, out_hbm, idx_v, ones_v, bounce_v, deg_sh, sem):
    c = lax.axis_index("c")
    s = lax.axis_index("s")
    w = s * NC + c

    zero16 = jnp.zeros((16,), jnp.float32)
    one16 = jnp.full((16,), 1.0, jnp.float32)

    def zrow(i, carry):
      bounce_v[i, :] = zero16
      return carry

    lax.fori_loop(0, RPT, zrow, 0)

    def orow(i, carry):
      ones_v[i, :] = one16
      return carry

    lax.fori_loop(0, CW, orow, 0)

    pltpu.sync_copy(bounce_v, deg_sh.at[pl.ds(s * RPT, RPT)])
    plsc.subcore_barrier()

    pltpu.sync_copy(dst_hbm.at[w], idx_v)
    for g in range(0, rows_per_w, 25):
      ds_ = [pltpu.async_copy(ones_v, deg_sh.at[idx_v.at[g + k]], sem,
                              add=True) for k in range(25)]
      for d in ds_:
        d.wait()

    plsc.subcore_barrier()
    pltpu.sync_copy(deg_sh.at[pl.ds(s * RPT, RPT)], bounce_v)
    pltpu.sync_copy(bounce_v, out_hbm.at[c].at[pl.ds(s * RPT, RPT)])

  return deg_kernel(dst3)


def _sc_scatter(src3, dst3, table):
  n, h = table.shape
  rows_per_w = src3.shape[1]
  ngrp = rows_per_w // RPC

  @functools.partial(
      pl.kernel,
      out_type=jax.ShapeDtypeStruct((NC, NP, h), jnp.float32),
      mesh=_mesh(),
      compiler_params=_SC_PARAMS,
      scratch_types=[
          pltpu.VMEM((rows_per_w, CW), jnp.int32),
          pltpu.VMEM((rows_per_w, CW), jnp.int32),
          [pltpu.VMEM((CW, h), jnp.float32) for _ in range(2 * RPC)],
          pltpu.VMEM((CW, h), jnp.float32),
          pltpu.VMEM_SHARED((NP, h), jnp.float32),
          pltpu.SemaphoreType.DMA,
          pltpu.SemaphoreType.DMA,
      ],
  )
  def scat_kernel(src_hbm, dst_hbm, tab_hbm, out_hbm, sidx_v, didx_v, rows_v,
                  bounce_v, acc_sh, gsem, ssem):
    c = lax.axis_index("c")
    s = lax.axis_index("s")
    w = s * NC + c

    zero16 = jnp.zeros((16,), jnp.float32)

    def zrow(i, carry):
      for j in range(h // 16):
        bounce_v[i, pl.ds(16 * j, 16)] = zero16
      return carry

    lax.fori_loop(0, CW, zrow, 0)
    for j in range(RPT // CW):
      pltpu.sync_copy(bounce_v, acc_sh.at[pl.ds(s * RPT + j * CW, CW)])
    plsc.subcore_barrier()

    pltpu.sync_copy(src_hbm.at[w], sidx_v)
    pltpu.sync_copy(dst_hbm.at[w], didx_v)

    pend = [None, None]
    for g in range(ngrp):
      bs = g % 2
      if pend[bs] is not None:
        for d in pend[bs]:
          d.wait()
      gds = []
      for j in range(RPC):
        r = g * RPC + j
        gds.append(pltpu.async_copy(
            tab_hbm.at[sidx_v.at[r]], rows_v[bs * RPC + j], gsem))
      for d in gds:
        d.wait()
      sds = []
      for j in range(RPC):
        r = g * RPC + j
        sds.append(pltpu.async_copy(
            rows_v[bs * RPC + j], acc_sh.at[didx_v.at[r]], ssem, add=True))
      pend[bs] = sds
    for p in pend:
      if p is not None:
        for d in p:
          d.wait()

    plsc.subcore_barrier()
    for j in range(RPT // CW):
      r = s * RPT + j * CW
      pltpu.sync_copy(acc_sh.at[pl.ds(r, CW)], bounce_v)
      pltpu.sync_copy(bounce_v, out_hbm.at[c].at[pl.ds(r, CW)])

  return scat_kernel(src3, dst3, table)


def _sc_gather2(src3, dst3, u, v):
  n, h = u.shape
  rows_per_w = src3.shape[1]
  e = NW * rows_per_w * CW
  ngrp = rows_per_w // RPC

  @functools.partial(
      pl.kernel,
      out_type=jax.ShapeDtypeStruct((e, 2 * h), jnp.float32),
      mesh=_mesh(),
      compiler_params=_SC_PARAMS,
      scratch_types=[
          pltpu.VMEM((rows_per_w, CW), jnp.int32),
          pltpu.VMEM((rows_per_w, CW), jnp.int32),
          [pltpu.VMEM((CW, h), jnp.float32) for _ in range(2 * RPC)],
          [pltpu.VMEM((CW, h), jnp.float32) for _ in range(2 * RPC)],
          pltpu.SemaphoreType.DMA,
          pltpu.SemaphoreType.DMA,
      ],
  )
  def gat_kernel(src_hbm, dst_hbm, u_hbm, v_hbm, g_hbm, sidx_v,
                 didx_v, ubuf_v, vbuf_v, gsem, wsem):
    c = lax.axis_index("c")
    s = lax.axis_index("s")
    w = s * NC + c

    pltpu.sync_copy(src_hbm.at[w], sidx_v)
    pltpu.sync_copy(dst_hbm.at[w], didx_v)

    pend = [None, None]
    for g in range(ngrp):
      bs = g % 2
      if pend[bs] is not None:
        for d in pend[bs]:
          d.wait()
      gds = []
      for j in range(RPC):
        r = g * RPC + j
        gds.append(pltpu.async_copy(
            u_hbm.at[sidx_v.at[r]], ubuf_v[bs * RPC + j], gsem))
        gds.append(pltpu.async_copy(
            v_hbm.at[didx_v.at[r]], vbuf_v[bs * RPC + j], gsem))
      for d in gds:
        d.wait()
      wds = []
      for j in range(RPC):
        o = (w * rows_per_w + g * RPC + j) * CW
        wds.append(pltpu.async_copy(
            ubuf_v[bs * RPC + j], g_hbm.at[pl.ds(o, CW), pl.ds(0, h)], wsem))
        wds.append(pltpu.async_copy(
            vbuf_v[bs * RPC + j], g_hbm.at[pl.ds(o, CW), pl.ds(h, h)], wsem))
      pend[bs] = wds
    for p in pend:
      if p is not None:
        for d in p:
          d.wait()

  return gat_kernel(src3, dst3, u, v)


def _dinv_from(degp_ref, n):
  deg = degp_ref[0:n, 0:1] + degp_ref[NP:NP + n, 0:1] + 1.0
  return 1.0 / jnp.sqrt(deg)


def _tc_embed(x, w0, b0, w1, degp):
  n = x.shape[0]
  h = w1.shape[1]

  def body(x_ref, w0_ref, b0_ref, w1_ref, degp_ref, out_ref):
    dinv = _dinv_from(degp_ref, n)
    w01 = jnp.dot(w0_ref[...], w1_ref[...], preferred_element_type=jnp.float32)
    b01 = jnp.dot(b0_ref[...], w1_ref[...], preferred_element_type=jnp.float32)
    t = jnp.dot(x_ref[...], w01, preferred_element_type=jnp.float32) + b01
    out_ref[...] = t * dinv

  return pl.pallas_call(
      body,
      out_shape=jax.ShapeDtypeStruct((n, h), jnp.float32),
  )(x, w0, b0.reshape(1, -1), w1, degp)


def _tc_layer(accp, table, degp, b, w_next):
  n, h = table.shape

  def body(accp_ref, tab_ref, degp_ref, b_ref, w_ref, out_ref):
    dinv = _dinv_from(degp_ref, n)
    acc = accp_ref[0:n, :] + accp_ref[NP:NP + n, :] + tab_ref[...]
    hh = jnp.maximum(acc * dinv + b_ref[...], 0.0)
    out_ref[...] = jnp.dot(hh, w_ref[...],
                           preferred_element_type=jnp.float32) * dinv

  return pl.pallas_call(
      body,
      out_shape=jax.ShapeDtypeStruct((n, h), jnp.float32),
  )(accp, table, degp, b.reshape(1, -1), w_next)


def _tc_head(accp, table, degp, b, wc1):
  n, h = table.shape

  def body(accp_ref, tab_ref, degp_ref, b_ref, wc1_ref, u_ref, v_ref):
    dinv = _dinv_from(degp_ref, n)
    acc = accp_ref[0:n, :] + accp_ref[NP:NP + n, :] + tab_ref[...]
    h3 = acc * dinv + b_ref[...]
    u_ref[...] = jnp.dot(h3, wc1_ref[0:h, :],
                         preferred_element_type=jnp.float32)
    v_ref[...] = jnp.dot(h3, wc1_ref[h:2 * h, :],
                         preferred_element_type=jnp.float32)

  return pl.pallas_call(
      body,
      out_shape=(jax.ShapeDtypeStruct((n, h), jnp.float32),
                 jax.ShapeDtypeStruct((n, h), jnp.float32)),
  )(accp, table, degp, b.reshape(1, -1), wc1)


def _tc_final(g, edge_attr, we, be, wc1, bc1, wc2, bc2):
  e = g.shape[0]
  h = g.shape[1] // 2
  fe = edge_attr.shape[1]
  blk = 8192
  grid = (e + blk - 1) // blk
  orows = blk // 128

  def body(g_ref, ea_ref, we_ref, be_ref, wc1e_ref, bc1_ref, wc2_ref,
           bc2_ref, out_ref):
    ef = jnp.maximum(
        jnp.dot(ea_ref[...], we_ref[...], preferred_element_type=jnp.float32)
        + be_ref[...], 0.0)
    t = jnp.dot(ef, wc1e_ref[...], preferred_element_type=jnp.float32)
    z = jnp.maximum(g_ref[:, 0:h] + g_ref[:, h:2 * h] + t + bc1_ref[...], 0.0)
    logit = jnp.dot(z, wc2_ref[...], preferred_element_type=jnp.float32) \
        + bc2_ref[...]
    lg = jnp.reshape(logit, (orows, 128))
    out_ref[...] = 1.0 / (1.0 + jnp.exp(-lg))

  return pl.pallas_call(
      body,
      grid=(grid,),
      in_specs=[
          pl.BlockSpec((blk, 2 * h), lambda i: (i, 0)),
          pl.BlockSpec((blk, fe), lambda i: (i, 0)),
          pl.BlockSpec((fe, h), lambda i: (0, 0)),
          pl.BlockSpec((1, h), lambda i: (0, 0)),
          pl.BlockSpec((h, h), lambda i: (0, 0)),
          pl.BlockSpec((1, h), lambda i: (0, 0)),
          pl.BlockSpec((h, 1), lambda i: (0, 0)),
          pl.BlockSpec((1, 1), lambda i: (0, 0)),
      ],
      out_specs=pl.BlockSpec((orows, 128), lambda i: (i, 0)),
      out_shape=jax.ShapeDtypeStruct((e // 128, 128), jnp.float32),
  )(g, edge_attr, we.reshape(fe, h), be.reshape(1, -1), wc1[2 * h:, :],
    bc1.reshape(1, -1), wc2, bc2.reshape(1, -1))


def kernel(x, edge_index, edge_attr, W0, b0, W1, b1, W2, b2, W3, b3, We, be,
           Wc1, bc1, Wc2, bc2):
  n = x.shape[0]
  e = edge_index.shape[1]
  rows_per_w = e // (NW * CW)
  ei = edge_index.astype(jnp.int32)
  src3 = ei[0].reshape(NW, rows_per_w, CW)
  dst3 = ei[1].reshape(NW, rows_per_w, CW)

  degp = _sc_degree(dst3).reshape(NC * NP, 16)
  t1 = _tc_embed(x, W0, b0, W1, degp)
  acc1 = _sc_scatter(src3, dst3, t1).reshape(NC * NP, -1)
  t2 = _tc_layer(acc1, t1, degp, b1, W2)
  acc2 = _sc_scatter(src3, dst3, t2).reshape(NC * NP, -1)
  t3 = _tc_layer(acc2, t2, degp, b2, W3)
  acc3 = _sc_scatter(src3, dst3, t3).reshape(NC * NP, -1)
  u, v = _tc_head(acc3, t3, degp, b3, Wc1)
  g = _sc_gather2(src3, dst3, u, v)
  p = _tc_final(g, edge_attr, We, be, Wc1, bc1, Wc2, bc2)
  return p.reshape(e)
